# baseline (speedup 1.0000x reference)
import jax
import jax.numpy as jnp
from jax import lax
from jax.experimental import pallas as pl
from jax.experimental.pallas import tpu as pltpu
from jax.experimental.pallas import tpu_sc as plsc

B = 16384
NC = 26
NN = 13
V = 100000
D = 32
NF = NC + NN

NCORES = 2
NSUB = 16
HB = 8192
VA = 99968


def _sc_body(tab2, xc1, xn1, rown, tail1, w1, b1, out,
             staged, idx_v, obuf0, obuf1, rown_v, wv, bvv, sem_g, sem_o):
    c = lax.axis_index("c")
    s = lax.axis_index("s")
    d = c * NSUB + s
    pltpu.sync_copy(w1, wv)
    pltpu.sync_copy(b1, bvv)
    pltpu.sync_copy(rown.at[d], rown_v)
    zero16 = jnp.zeros((16,), dtype=jnp.int32)
    stage = [None]
    prevh = [None, None]

    def fire_stage(f):
        stage[0] = pltpu.async_copy(
            tab2.at[rown_v.at[f], pl.ds(0, VA)], staged.at[:, pl.ds(0, VA)],
            sem_g)
        pltpu.sync_copy(tail1.at[pl.ds(d * (NC * D) + f * D, D)],
                        staged.at[0, pl.ds(VA, D)])

    def half_unit(f, h, buf, preloaded=False):
        obuf = obuf0 if buf == 0 else obuf1
        if f < NC:
            if not preloaded:
                pltpu.sync_copy(xc1.at[pl.ds(f * B + h * HB, HB)], idx_v)
            wsp = bsp = None
        else:
            pltpu.sync_copy(xn1.at[pl.ds((f - NC) * B + h * HB, HB)], idx_v)
            sel = jnp.full((16,), d * NN + (f - NC), dtype=jnp.int32)
            wsp = plsc.load_gather(wv, [sel])
            bsp = plsc.load_gather(bvv, [sel])
        if prevh[buf] is not None:
            prevh[buf].wait()
        if f < NC:
            def gbody(t, _):
                for k in range(4):
                    vi = idx_v[pl.ds(t * 64 + k * 16, 16)]
                    obuf[0, pl.ds(t * 64 + k * 16, 16)] = (
                        plsc.load_gather(staged, [zero16, vi]))
                return 0
            lax.fori_loop(0, HB // 64, gbody, 0)
        else:
            def nbody(t, _):
                for k in range(4):
                    vx = plsc.bitcast(
                        idx_v[pl.ds(t * 64 + k * 16, 16)], jnp.float32)
                    obuf[0, pl.ds(t * 64 + k * 16, 16)] = vx * wsp + bsp
                return 0
            lax.fori_loop(0, HB // 64, nbody, 0)
        prevh[buf] = pltpu.async_copy(
            obuf, out.at[rown_v.at[f], pl.ds(h * HB, HB)], sem_o)

    fillers = [(NC + j, h) for j in range(NN) for h in range(2)]
    fire_stage(0)
    pltpu.sync_copy(xc1.at[pl.ds(0, HB)], idx_v)
    nbuf = 0
    for f in range(NC):
        stage[0].wait()
        half_unit(f, 0, nbuf, preloaded=True)
        nbuf ^= 1
        half_unit(f, 1, nbuf)
        nbuf ^= 1
        if f + 1 < NC:
            fire_stage(f + 1)
        nf, nh = fillers[f]
        half_unit(nf, nh, nbuf)
        nbuf ^= 1
        if f + 1 < NC:
            pltpu.sync_copy(xc1.at[pl.ds((f + 1) * B, HB)], idx_v)
    for buf in range(2):
        if prevh[buf] is not None:
            prevh[buf].wait()


@jax.jit
def _tokenize(tab2, xc1, xn1, rown, tail1, w1, b1):
    return pl.kernel(
        _sc_body,
        out_type=jax.ShapeDtypeStruct((NF * D, B), jnp.float32),
        mesh=plsc.VectorSubcoreMesh(core_axis_name="c", subcore_axis_name="s",
                                    num_cores=NCORES, num_subcores=NSUB),
        compiler_params=pltpu.CompilerParams(needs_layout_passes=False,
                                             use_tc_tiling_on_sc=True),
        scratch_types=[
            pltpu.VMEM((1, V), jnp.float32),
            pltpu.VMEM((HB,), jnp.int32),
            pltpu.VMEM((1, HB), jnp.float32),
            pltpu.VMEM((1, HB), jnp.float32),
            pltpu.VMEM((40, 1), jnp.int32),
            pltpu.VMEM((NN * D,), jnp.float32),
            pltpu.VMEM((NN * D,), jnp.float32),
            pltpu.SemaphoreType.DMA,
            pltpu.SemaphoreType.DMA,
        ],
    )(tab2, xc1, xn1, rown, tail1, w1, b1)


def kernel(x_num, x_cat, cat_tables, num_W, num_b):
    tab2 = cat_tables.transpose(0, 2, 1).reshape(NC * D, V)
    xc1 = x_cat.astype(jnp.int32).T.reshape(NC * B)
    xn1 = lax.bitcast_convert_type(x_num, jnp.int32).T.reshape(NN * B)
    w1 = num_W.T.reshape(NN * D)
    b1 = num_b.T.reshape(NN * D)
    rown = (jnp.arange(D, dtype=jnp.int32)[:, None]
            + jnp.arange(NF + 1, dtype=jnp.int32)[None, :] * D)
    rown = rown.reshape(D, NF + 1, 1)
    tail1 = cat_tables[:, VA:, :].transpose(2, 0, 1).reshape(D * NC * D)
    outp = _tokenize(tab2, xc1, xn1, rown, tail1, w1, b1)
    return outp.reshape(NF, D, B).transpose(2, 0, 1)
